# Initial kernel scaffold; baseline (speedup 1.0000x reference)
import functools

import jax
import jax.numpy as jnp
from jax import lax
from jax.experimental import pallas as pl
from jax.experimental.pallas import tpu as pltpu
from jax.experimental.pallas import tpu_sc as plsc


def _gather_kernel(n, D, NC, NS, C):
    NW = NC * NS
    n_w = n // NW
    n_chunks = n_w // C
    mesh = plsc.VectorSubcoreMesh(core_axis_name="c", subcore_axis_name="s")

    @functools.partial(
        pl.kernel,
        mesh=mesh,
        out_type=jax.ShapeDtypeStruct((n, D), jnp.float32),
        scratch_types=[
            pltpu.VMEM((n_chunks, C), jnp.int32),
            pltpu.VMEM((C, D), jnp.float32),
            pltpu.SemaphoreType.DMA,
        ],
    )
    def k(table_hbm, x_hbm, out_hbm, idx_v, rows_v, sem):
        wid = lax.axis_index("s") * NC + lax.axis_index("c")
        pltpu.sync_copy(x_hbm.at[wid], idx_v)

        def body(j, carry):
            pltpu.async_copy(table_hbm.at[idx_v.at[j]], rows_v, sem).wait()
            pltpu.sync_copy(rows_v, out_hbm.at[pl.ds(wid * n_w + j * C, C)])
            return carry

        lax.fori_loop(0, n_chunks, body, 0)

    return k


def kernel(table, x):
    V, D = table.shape
    B, L = x.shape
    n = B * L
    info = plsc.get_sparse_core_info()
    NC, NS = info.num_cores, info.num_subcores
    NW = NC * NS
    C = 128
    assert n % (NW * C) == 0
    xf = x.reshape(NW, n // (NW * C), C).astype(jnp.int32)
    out = _gather_kernel(n, D, NC, NS, C)(table, xf)
    return out.reshape(B, L, D)


# speedup vs baseline: 1.3070x; 1.3070x over previous
import functools

import jax
import jax.numpy as jnp
from jax import lax
from jax.experimental import pallas as pl
from jax.experimental.pallas import tpu as pltpu
from jax.experimental.pallas import tpu_sc as plsc


def _gather_kernel(n, D, NC, NS, C):
    NW = NC * NS
    n_w = n // NW
    n_chunks = n_w // C
    mesh = plsc.VectorSubcoreMesh(core_axis_name="c", subcore_axis_name="s")

    @functools.partial(
        pl.kernel,
        mesh=mesh,
        out_type=jax.ShapeDtypeStruct((n, D), jnp.float32),
        scratch_types=[
            pltpu.VMEM((n_chunks, C), jnp.int32),
            pltpu.VMEM((C, D), jnp.float32),
            pltpu.SemaphoreType.DMA,
        ],
        compiler_params=pltpu.CompilerParams(use_tc_tiling_on_sc=False),
    )
    def k(table_hbm, x_hbm, out_hbm, idx_v, rows_v, sem):
        wid = lax.axis_index("s") * NC + lax.axis_index("c")
        pltpu.sync_copy(x_hbm.at[wid], idx_v)

        def body(j, carry):
            pltpu.async_copy(table_hbm.at[idx_v.at[j]], rows_v, sem).wait()
            pltpu.sync_copy(rows_v, out_hbm.at[pl.ds(wid * n_w + j * C, C)])
            return carry

        lax.fori_loop(0, n_chunks, body, 0)

    return k


def kernel(table, x):
    V, D = table.shape
    B, L = x.shape
    n = B * L
    info = plsc.get_sparse_core_info()
    NC, NS = info.num_cores, info.num_subcores
    NW = NC * NS
    C = 128
    assert n % (NW * C) == 0
    xf = x.reshape(NW, n // (NW * C), C).astype(jnp.int32)
    out = _gather_kernel(n, D, NC, NS, C)(table, xf)
    return out.reshape(B, L, D)


# speedup vs baseline: 1.4995x; 1.1473x over previous
import functools

import jax
import jax.numpy as jnp
from jax import lax
from jax.experimental import pallas as pl
from jax.experimental.pallas import tpu as pltpu
from jax.experimental.pallas import tpu_sc as plsc


def _gather_kernel(n, D, NC, NS, C, NBUF):
    NW = NC * NS
    n_w = n // NW
    n_chunks = n_w // C
    n_rounds = n_chunks // NBUF
    mesh = plsc.VectorSubcoreMesh(core_axis_name="c", subcore_axis_name="s")

    @functools.partial(
        pl.kernel,
        mesh=mesh,
        out_type=jax.ShapeDtypeStruct((n, D), jnp.float32),
        scratch_types=(
            [pltpu.VMEM((n_chunks, C), jnp.int32),
             pltpu.VMEM((NBUF, C, D), jnp.float32)]
            + [pltpu.SemaphoreType.DMA] * (2 * NBUF)
        ),
        compiler_params=pltpu.CompilerParams(use_tc_tiling_on_sc=False),
    )
    def k(table_hbm, x_hbm, out_hbm, idx_v, rows_v, *sems):
        gsem = sems[:NBUF]
        osem = sems[NBUF:]
        wid = lax.axis_index("s") * NC + lax.axis_index("c")
        base = wid * n_w
        pltpu.sync_copy(x_hbm.at[wid], idx_v)

        for b in range(NBUF):
            pltpu.async_copy(table_hbm.at[idx_v.at[b]], rows_v.at[b], gsem[b])

        def round_body(r, carry):
            s0 = r * NBUF
            for b in range(NBUF):
                pltpu.make_async_copy(
                    table_hbm.at[idx_v.at[b]], rows_v.at[b], gsem[b]).wait()
                pltpu.async_copy(
                    rows_v.at[b],
                    out_hbm.at[pl.ds(base + (s0 + b) * C, C)],
                    osem[b])
            for b in range(NBUF):
                pltpu.make_async_copy(
                    rows_v.at[b],
                    out_hbm.at[pl.ds(base + (s0 + b) * C, C)],
                    osem[b]).wait()

                @pl.when(r < n_rounds - 1)
                def _():
                    pltpu.async_copy(
                        table_hbm.at[idx_v.at[s0 + NBUF + b]],
                        rows_v.at[b], gsem[b])
            return carry

        lax.fori_loop(0, n_rounds, round_body, 0)

    return k


def kernel(table, x):
    V, D = table.shape
    B, L = x.shape
    n = B * L
    info = plsc.get_sparse_core_info()
    NC, NS = info.num_cores, info.num_subcores
    NW = NC * NS
    C = 128
    NBUF = 8
    assert n % (NW * C * NBUF) == 0
    xf = x.reshape(NW, n // (NW * C), C).astype(jnp.int32)
    out = _gather_kernel(n, D, NC, NS, C, NBUF)(table, xf)
    return out.reshape(B, L, D)
